# Initial kernel scaffold; baseline (speedup 1.0000x reference)
import jax
import jax.numpy as jnp
from jax.experimental import pallas as pl


def kernel(edge_index, edge_values, X, W_pass, b_pass, W_self, b_self):
    raise NotImplementedError("write your pallas kernel here")


# speedup vs baseline: 5.4545x; 5.4545x over previous
import functools

import jax
import jax.numpy as jnp
from jax import lax
from jax.experimental import pallas as pl
from jax.experimental.pallas import tpu as pltpu
from jax.experimental.pallas import tpu_sc as plsc

N_NODES = 10000
N_EDGES = 320000
D = 128

NC = 2
NS = 16
NW = NC * NS

C = 128
CHUNKS = N_EDGES // C
FULL_ROUNDS = CHUNKS // NW
TAIL = CHUNKS - FULL_ROUNDS * NW
R_BASE = 624
ZR = 16


def _sc_body(rows_hbm, cols_hbm, vals_hbm, x_hbm, out_hbm,
             cols_v, rows_v, vals_v, gath_v, zero_v, acc, sem):
    c = lax.axis_index("c")
    s = lax.axis_index("s")
    wid = s * NC + c

    zeros16 = jnp.zeros((16,), jnp.float32)
    for r in range(ZR):
        for j in range(D // 16):
            zero_v[r, pl.ds(j * 16, 16)] = zeros16

    def zloop(i, carry):
        pltpu.sync_copy(zero_v, acc.at[pl.ds(s * R_BASE + i * ZR, ZR)])
        return carry

    n_zero = R_BASE // ZR + (s == NS - 1).astype(jnp.int32)
    lax.fori_loop(0, n_zero, zloop, 0)
    plsc.subcore_barrier()

    def echunk(k, carry):
        base = (wid + k * NW) * C
        pltpu.sync_copy(cols_hbm.at[pl.ds(base, C)], cols_v)
        pltpu.sync_copy(rows_hbm.at[pl.ds(base, C)], rows_v)
        pltpu.sync_copy(vals_hbm.at[pl.ds(base, C)], vals_v)
        pltpu.async_copy(x_hbm.at[cols_v], gath_v, sem).wait()

        def scale(g, inner):
            vv = vals_v[pl.ds(g * 16, 16)]
            for l in range(16):
                v = vv[l]
                e = g * 16 + l
                for j in range(D // 16):
                    sl = pl.ds(j * 16, 16)
                    gath_v[e, sl] = gath_v[e, sl] * v
            return inner

        lax.fori_loop(0, C // 16, scale, 0)
        pltpu.sync_copy(gath_v, acc.at[rows_v], add=True)
        return carry

    nch = FULL_ROUNDS + (wid < TAIL).astype(jnp.int32)
    lax.fori_loop(0, nch, echunk, 0)
    plsc.subcore_barrier()

    pltpu.sync_copy(acc.at[pl.ds(s * R_BASE, R_BASE)],
                    out_hbm.at[c, pl.ds(s * R_BASE, R_BASE)])

    @pl.when(s == NS - 1)
    def _tail_out():
        t = NS * R_BASE
        pltpu.sync_copy(acc.at[pl.ds(t, N_NODES - t)],
                        out_hbm.at[c, pl.ds(t, N_NODES - t)])


def _gcn_sc_partials(rows, cols, vals, x):
    mesh = plsc.VectorSubcoreMesh(core_axis_name="c", subcore_axis_name="s")
    kfn = pl.kernel(
        _sc_body,
        out_type=jax.ShapeDtypeStruct((NC, N_NODES, D), jnp.float32),
        mesh=mesh,
        scratch_types=[
            pltpu.VMEM((C,), jnp.int32),
            pltpu.VMEM((C,), jnp.int32),
            pltpu.VMEM((C,), jnp.float32),
            pltpu.VMEM((C, D), jnp.float32),
            pltpu.VMEM((ZR, D), jnp.float32),
            pltpu.VMEM_SHARED((N_NODES, D), jnp.float32),
            pltpu.SemaphoreType.DMA,
        ],
    )
    return kfn(rows, cols, vals, x)


def _tc_body(p_ref, x_ref, wp_ref, ws_ref, b_ref, o_ref):
    agg = p_ref[0] + p_ref[1]
    o_ref[...] = (
        jnp.dot(agg, wp_ref[...], preferred_element_type=jnp.float32)
        + jnp.dot(x_ref[...], ws_ref[...], preferred_element_type=jnp.float32)
        + b_ref[...]
    )


def _gcn_tc_combine(p, x, wp_t, ws_t, b):
    BR = 1000
    return pl.pallas_call(
        _tc_body,
        grid=(N_NODES // BR,),
        in_specs=[
            pl.BlockSpec((NC, BR, D), lambda i: (0, i, 0)),
            pl.BlockSpec((BR, D), lambda i: (i, 0)),
            pl.BlockSpec((D, D), lambda i: (0, 0)),
            pl.BlockSpec((D, D), lambda i: (0, 0)),
            pl.BlockSpec((1, D), lambda i: (0, 0)),
        ],
        out_specs=pl.BlockSpec((BR, D), lambda i: (i, 0)),
        out_shape=jax.ShapeDtypeStruct((N_NODES, D), jnp.float32),
    )(p, x, wp_t, ws_t, b)


@jax.jit
def _impl(edge_index, edge_values, X, W_pass, b_pass, W_self, b_self):
    rows = edge_index[0].astype(jnp.int32)
    cols = edge_index[1].astype(jnp.int32)
    p = _gcn_sc_partials(rows, cols, edge_values, X)
    b = (b_pass + b_self).reshape(1, D)
    return _gcn_tc_combine(p, X, W_pass.T, W_self.T, b)


def kernel(edge_index, edge_values, X, W_pass, b_pass, W_self, b_self):
    return _impl(edge_index, edge_values, X, W_pass, b_pass, W_self, b_self)
